# baseline (speedup 1.0000x reference)
import functools

import jax
import jax.numpy as jnp
from jax.experimental import pallas as pl
from jax.experimental.pallas import tpu as pltpu


def _mm_kernel(x_ref, w_ref, o_ref):
    o_ref[...] = jnp.dot(x_ref[...], w_ref[...],
                         preferred_element_type=jnp.float32)


def _matmul(x, w, block_m=1000):
    m, k = x.shape
    _, n = w.shape
    grid = (m // block_m,)
    return pl.pallas_call(
        _mm_kernel,
        grid=grid,
        in_specs=[
            pl.BlockSpec((block_m, k), lambda i: (i, 0)),
            pl.BlockSpec((k, n), lambda i: (0, 0)),
        ],
        out_specs=pl.BlockSpec((block_m, n), lambda i: (i, 0)),
        out_shape=jax.ShapeDtypeStruct((m, n), jnp.float32),
    )(x, w)


def kernel(node_features, edge_index, perturbation_indices, W_in, b_in,
           Wl0, Wr0, att0, bc0, g0, be0,
           Wl1, Wr1, att1, bc1, g1, be1,
           Wl2, Wr2, att2, bc2, g2, be2,
           A1, ab1, A2, ab2, Wo, bo):
    Nn = node_features.shape[0]
    src, dst = edge_index[0], edge_index[1]
    loop = jnp.arange(Nn, dtype=src.dtype)
    src_sl = jnp.concatenate([src, loop])
    dst_sl = jnp.concatenate([dst, loop])
    layers = [(Wl0, Wr0, att0, bc0, g0, be0, 8),
              (Wl1, Wr1, att1, bc1, g1, be1, 8),
              (Wl2, Wr2, att2, bc2, g2, be2, 1)]

    def conv(x, Wl, Wr, att, b, H):
        dh = Wl.shape[1] // H
        xl = _matmul(x, Wl).reshape(Nn, H, dh)
        xr = _matmul(x, Wr).reshape(Nn, H, dh)
        z = jax.nn.leaky_relu(xl[src_sl] + xr[dst_sl], 0.2)
        e = jnp.einsum('ehd,hd->eh', z, att)
        m = jax.ops.segment_max(e, dst_sl, num_segments=Nn)
        m = jnp.where(jnp.isfinite(m), m, 0.0)
        ex = jnp.exp(e - m[dst_sl])
        den = jax.ops.segment_sum(ex, dst_sl, num_segments=Nn)
        alpha = ex / (den[dst_sl] + 1e-16)
        out = jax.ops.segment_sum(xl[src_sl] * alpha[:, :, None], dst_sl,
                                  num_segments=Nn)
        return out.reshape(Nn, -1) + b

    def ln(v, g, b):
        mu = v.mean(-1, keepdims=True)
        var = ((v - mu) ** 2).mean(-1, keepdims=True)
        return (v - mu) / jnp.sqrt(var + 1e-5) * g + b

    def encoder(x):
        h = jax.nn.elu(_matmul(x, W_in) + b_in)
        for (Wl, Wr, att, bc, g, be, H) in layers:
            o = jax.nn.elu(conv(h, Wl, Wr, att, bc, H))
            h = ln(o + h, g, be)
        return h

    def bfs_mask(p):
        visited = jnp.zeros((Nn,), dtype=jnp.float32).at[p].set(1.0)
        frontier = visited
        for _ in range(2):
            reach = jnp.zeros((Nn,), dtype=jnp.float32).at[dst].max(frontier[src])
            visited = jnp.maximum(visited, reach)
            frontier = reach
        return visited > 0

    def pert_embed(p):
        x = node_features.at[p].set(0.0)
        h = encoder(x)
        mask = bfs_mask(p)
        s = (jnp.tanh(_matmul(h, A1) + ab1) @ A2 + ab2)[:, 0]
        s = jnp.where(mask, s, -1e30)
        a = jax.nn.softmax(s)
        agg = a @ h
        return agg @ Wo + bo

    outs = [pert_embed(perturbation_indices[i])
            for i in range(perturbation_indices.shape[0])]
    return jnp.stack(outs, axis=0)
